# Initial kernel scaffold; baseline (speedup 1.0000x reference)
import jax
import jax.numpy as jnp
from jax.experimental import pallas as pl


def kernel(input):
    raise NotImplementedError("write your pallas kernel here")


# speedup vs baseline: 2.8185x; 2.8185x over previous
import functools

import jax
import jax.numpy as jnp
from jax import lax
from jax.experimental import pallas as pl
from jax.experimental.pallas import tpu as pltpu
from jax.experimental.pallas import tpu_sc as plsc

B, C, H, W = 32, 768, 32, 32
N = H * W
R = B * C
K = 205
ALPHA = 0.7

NC, NS, L = 2, 16, 16
NW = NC * NS
RPW = R // NW
GROUP = 16
NGRP = RPW // GROUP
CHUNKS = N // L

MIN32 = -2147483648


def _kernel_body(x_hbm, out_hbm, xbuf, kbuf, outbuf):
    wid = lax.axis_index("s") * NC + lax.axis_index("c")
    zero = jnp.zeros((L,), jnp.int32)
    one = jnp.ones((L,), jnp.int32)
    fzero = jnp.zeros((L,), jnp.float32)
    lanes = lax.iota(jnp.int32, L)

    def group_body(g, carry):
        row0 = wid * RPW + g * GROUP
        pltpu.sync_copy(x_hbm.at[pl.ds(row0 * N, GROUP * N)], xbuf)

        def key_body(j, _):
            for u in range(8):
                off = j * (8 * L) + u * L
                v = xbuf[pl.ds(off, L)]
                b = plsc.bitcast(v, jnp.int32)
                ks = jnp.where(b >= 0, b, MIN32 - b)
                kbuf[pl.ds(off, L)] = (ks >> 16) + 32768
            return 0

        lax.fori_loop(0, GROUP * CHUNKS // 8, key_body, 0)

        def row_body(r, ovec):
            base = r * N

            def count_pass(cand1, cand2p):
                def cbody(j, c):
                    c1a, c1b, c2a, c2b = c
                    for u in range(8):
                        v = kbuf[pl.ds(base + j * (8 * L) + u * L, L)]
                        i1 = jnp.where(v >= cand1, one, zero)
                        i2 = jnp.where(v <= cand2p, one, zero)
                        if u % 2 == 0:
                            c1a = c1a + i1
                            c2a = c2a + i2
                        else:
                            c1b = c1b + i1
                            c2b = c2b + i2
                    return c1a, c1b, c2a, c2b

                c1a, c1b, c2a, c2b = lax.fori_loop(
                    0, CHUNKS // 8, cbody, (zero, zero, zero, zero))
                return jnp.sum(c1a + c1b), jnp.sum(c2a + c2b)

            def bit_body(i, st):
                t1, t2, bit = st
                cand1 = t1 + bit
                cand2 = t2 + bit
                n1, n2 = count_pass(cand1, jnp.int32(65535) - cand2)
                t1 = jnp.where(n1 >= K, cand1, t1)
                t2 = jnp.where(n2 >= K, cand2, t2)
                return t1, t2, bit >> 1

            t1, t2, _ = lax.fori_loop(
                0, 16, bit_body,
                (jnp.int32(0), jnp.int32(0), jnp.int32(32768)))
            thr_top = t1
            thr_bot = jnp.int32(65535) - t2

            def fbody(j, c):
                cg, sg, cl, sl = c
                for u in range(8):
                    off = base + j * (8 * L) + u * L
                    v = kbuf[pl.ds(off, L)]
                    xv = xbuf[pl.ds(off, L)]
                    m1 = v > thr_top
                    m2 = v < thr_bot
                    cg = cg + jnp.where(m1, one, zero)
                    sg = sg + jnp.where(m1, xv, fzero)
                    cl = cl + jnp.where(m2, one, zero)
                    sl = sl + jnp.where(m2, xv, fzero)
                return cg, sg, cl, sl

            cg, sg, cl, sl = lax.fori_loop(
                0, CHUNKS // 8, fbody, (zero, fzero, zero, fzero))

            def dekey_vec(key16):
                ks_rep = ((key16 - 32768) << 16) + 32768
                kv = jnp.full((L,), ks_rep, jnp.int32)
                bv = jnp.where(kv >= 0, kv, MIN32 - kv)
                return plsc.bitcast(bv, jnp.float32)

            val_top = dekey_vec(thr_top)
            val_bot = dekey_vec(thr_bot)
            ng = jnp.full((L,), K - jnp.sum(cg), jnp.int32).astype(jnp.float32)
            nl = jnp.full((L,), K - jnp.sum(cl), jnp.int32).astype(jnp.float32)
            sgv = jnp.full((L,), jnp.sum(sg), jnp.float32)
            slv = jnp.full((L,), jnp.sum(sl), jnp.float32)
            top_sum = sgv + ng * val_top
            bot_sum = slv + nl * val_bot
            outv = top_sum * (1.0 / (2 * K)) + bot_sum * (ALPHA / (2 * K))
            return jnp.where(lanes == r, outv, ovec)

        ovec = lax.fori_loop(0, GROUP, row_body, fzero)
        outbuf[pl.ds(g * GROUP, GROUP)] = ovec
        return carry

    lax.fori_loop(0, NGRP, group_body, 0)
    pltpu.sync_copy(outbuf, out_hbm.at[pl.ds(wid * RPW, RPW)])


@jax.jit
def kernel(input):
    x = input.reshape(R * N)
    mesh = plsc.VectorSubcoreMesh(
        core_axis_name="c", subcore_axis_name="s",
        num_cores=NC, num_subcores=NS)
    out = pl.kernel(
        _kernel_body,
        out_type=jax.ShapeDtypeStruct((R,), jnp.float32),
        mesh=mesh,
        compiler_params=pltpu.CompilerParams(needs_layout_passes=False),
        scratch_types=[
            pltpu.VMEM((GROUP * N,), jnp.float32),
            pltpu.VMEM((GROUP * N,), jnp.int32),
            pltpu.VMEM((RPW,), jnp.float32),
        ],
    )(x)
    return out.reshape(B, C)
